# baseline (speedup 1.0000x reference)
import functools

import jax
import jax.numpy as jnp
from jax import lax
from jax.experimental import pallas as pl
from jax.experimental.pallas import tpu as pltpu
from jax.experimental.pallas import tpu_sc as plsc

B, P, H, W = 16, 4, 64, 128
N = B * P * H * W
D = 256
NC, NS = 2, 16
NW = NC * NS
NT = N // NW
C = 64
IB = 4096
CB = IB // C
QB = CB // 4
NBLK = NT // IB
G = NT // C
GQ = G // 4
R2 = 80
R1 = 300


def _prep_body(c_ref, n_ref, l_ref, num_ref, p_ref, w_ref, dk_ref, b_ref,
               t1_ref, t2_ref):
    dk = dk_ref[...]
    t1_ref[...] = jnp.dot(c_ref[...], dk[0:64, :],
                          preferred_element_type=jnp.float32)
    n_proj = jnp.dot(n_ref[...], dk[64:80, :],
                     preferred_element_type=jnp.float32)
    l_proj = jnp.dot(l_ref[...], dk[80:112, :],
                     preferred_element_type=jnp.float32)
    num_proj = jnp.dot(num_ref[...], dk[112:144, :],
                       preferred_element_type=jnp.float32)
    p_proj = jnp.dot(p_ref[...], dk[144:176, :],
                     preferred_element_type=jnp.float32)
    w_proj = jnp.dot(w_ref[...], dk[176:208, :],
                     preferred_element_type=jnp.float32)

    idx = lax.broadcasted_iota(jnp.int32, (R2, 1), 0)
    nb = idx // 16
    lbit = (idx // 8) % 2
    nbit = (idx // 4) % 2
    pbit = (idx // 2) % 2
    wbit = idx % 2

    acc = b_ref[...]
    for k in range(5):
        acc = acc + jnp.where(nb == k, 1.0, 0.0) * n_proj[k:k + 1, :]
    acc = acc + jnp.where(lbit == 1, l_proj[1:2, :], l_proj[0:1, :])
    acc = acc + jnp.where(nbit == 1, num_proj[1:2, :], num_proj[0:1, :])
    acc = acc + jnp.where(pbit == 1, p_proj[1:2, :], p_proj[0:1, :])
    acc = acc + jnp.where(wbit == 1, w_proj[1:2, :], w_proj[0:1, :])
    t2_ref[...] = acc


def _prep_tables(c_table, n_table, l_table, num_table, p_table, w_table,
                 dense_kernel, dense_bias):
    return pl.pallas_call(
        _prep_body,
        out_shape=[
            jax.ShapeDtypeStruct((R1, D), jnp.float32),
            jax.ShapeDtypeStruct((R2, D), jnp.float32),
        ],
    )(c_table, n_table, l_table, num_table, p_table, w_table,
      dense_kernel, dense_bias.reshape(1, D))


def _sum_body(t1_ref, t2_ref, t12_ref):
    t12_ref[...] = t2_ref[...][:, None, :] + t1_ref[...][None, :, :]


def _sum_tables(t1, t2):
    return pl.pallas_call(
        _sum_body,
        grid=(R2 // 8,),
        in_specs=[
            pl.BlockSpec((R1, D), lambda j: (0, 0)),
            pl.BlockSpec((8, D), lambda j: (j, 0)),
        ],
        out_specs=pl.BlockSpec((8, R1, D), lambda j: (j, 0, 0)),
        out_shape=jax.ShapeDtypeStruct((R2, R1, D), jnp.float32),
    )(t1, t2)


def _sc_body(cc, nb, il, inum, ip, iw, t12, out,
             cc_v, nb_v, il_v, in_v, ip_v, iw_v, i_b,
             buf0, buf1, buf2, buf3, semIdx,
             g0, g1, g2, g3, o0, o1, o2, o3):
    wid = lax.axis_index("s") * NC + lax.axis_index("c")
    base0 = wid * NT

    def issue_gather(cl, buf, gsem):
        pltpu.async_copy(t12.at[i_b.at[pl.ds(cl * C, C)]], buf, gsem)

    def wait_gather(buf, gsem):
        pltpu.make_async_copy(t12.at[i_b.at[pl.ds(0, C)]], buf, gsem).wait()

    def writeout(cl, blk, buf, osem):
        base = base0 + blk * IB + cl * C
        pltpu.async_copy(buf, out.at[pl.ds(base, C)], osem)

    def wait_out(buf, osem):
        pltpu.make_async_copy(buf, out.at[pl.ds(0, C)], osem).wait()

    FULL = ((buf0, g0, o0), (buf1, g1, o1), (buf2, g2, o2), (buf3, g3, o3))
    SLOTS = tuple((b, o) for b, _, o in FULL)

    def iblock(blk, carry):
        bbase = base0 + blk * IB
        cps = [pltpu.async_copy(src.at[pl.ds(bbase, IB)], dst, semIdx)
               for src, dst in zip((cc, nb, il, inum, ip, iw),
                                   (cc_v, nb_v, il_v, in_v, ip_v, iw_v))]
        for cp in cps:
            cp.wait()

        ioff = blk * IB

        def ixbody(j, carry2):
            sl = pl.ds(j * 16, 16)
            i_b[pl.ds(ioff + j * 16, 16)] = \
                (nb_v[sl] * 16 + il_v[sl] * 8 + in_v[sl] * 4
                 + ip_v[sl] * 2 + iw_v[sl]) * R1 + lax.rem(cc_v[sl], R1)
            return carry2

        lax.fori_loop(0, IB // 16, ixbody, 0)
        return carry

    lax.fori_loop(0, NBLK, iblock, 0)

    for m, (buf, gsem, osem) in enumerate(FULL):
        issue_gather(m, buf, gsem)

    def quad(q, carry2):
        c0 = 4 * q
        for m, (buf, gsem, osem) in enumerate(FULL):
            wait_gather(buf, gsem)
            writeout(c0 + m, 0, buf, osem)

        @pl.when(q < GQ - 1)
        def _():
            for m, (buf, gsem, osem) in enumerate(FULL):
                wait_out(buf, osem)
                issue_gather(c0 + 4 + m, buf, gsem)

        return carry2

    lax.fori_loop(0, GQ, quad, 0)
    for buf, osem in SLOTS:
        wait_out(buf, osem)


_sc_kernel = functools.partial(
    pl.kernel,
    mesh=plsc.VectorSubcoreMesh(core_axis_name="c", subcore_axis_name="s"),
    out_type=jax.ShapeDtypeStruct((N, D), jnp.float32),
    scratch_types=[
        pltpu.VMEM((IB,), jnp.int32),
        pltpu.VMEM((IB,), jnp.int32),
        pltpu.VMEM((IB,), jnp.int32),
        pltpu.VMEM((IB,), jnp.int32),
        pltpu.VMEM((IB,), jnp.int32),
        pltpu.VMEM((IB,), jnp.int32),
        pltpu.VMEM((NT,), jnp.int32),
        pltpu.VMEM((C, D), jnp.float32),
        pltpu.VMEM((C, D), jnp.float32),
        pltpu.VMEM((C, D), jnp.float32),
        pltpu.VMEM((C, D), jnp.float32),
        pltpu.SemaphoreType.DMA,
        pltpu.SemaphoreType.DMA,
        pltpu.SemaphoreType.DMA,
        pltpu.SemaphoreType.DMA,
        pltpu.SemaphoreType.DMA,
        pltpu.SemaphoreType.DMA,
        pltpu.SemaphoreType.DMA,
        pltpu.SemaphoreType.DMA,
        pltpu.SemaphoreType.DMA,
    ],
)(_sc_body)


def kernel(char_code, num_bytes, is_letter, is_number, is_punctuation,
           is_whitespace, c_table, n_table, l_table, num_table, p_table,
           w_table, dense_kernel, dense_bias):
    t1, t2 = _prep_tables(c_table, n_table, l_table, num_table, p_table,
                          w_table, dense_kernel, dense_bias)
    t12 = _sum_tables(t1, t2).reshape(R2 * R1, D)
    cc = char_code.reshape(N).astype(jnp.int32)
    nb = num_bytes.reshape(N).astype(jnp.int32)
    il = is_letter.reshape(N).astype(jnp.int32)
    inum = is_number.reshape(N).astype(jnp.int32)
    ip = is_punctuation.reshape(N).astype(jnp.int32)
    iw = is_whitespace.reshape(N).astype(jnp.int32)
    out = _sc_kernel(cc, nb, il, inum, ip, iw, t12)
    return out.reshape(B, P, H, W, D)


# speedup vs baseline: 1.0677x; 1.0677x over previous
import functools

import jax
import jax.numpy as jnp
from jax import lax
from jax.experimental import pallas as pl
from jax.experimental.pallas import tpu as pltpu
from jax.experimental.pallas import tpu_sc as plsc

B, P, H, W = 16, 4, 64, 128
N = B * P * H * W
D = 256
NC, NS = 2, 16
NW = NC * NS
NT = N // NW
C = 64
IB = 4096
CB = IB // C
QB = CB // 4
NBLK = NT // IB
R2 = 80
R1 = 300


def _prep_body(c_ref, n_ref, l_ref, num_ref, p_ref, w_ref, dk_ref, b_ref,
               t1_ref, t2_ref):
    dk = dk_ref[...]
    t1_ref[...] = jnp.dot(c_ref[...], dk[0:64, :],
                          preferred_element_type=jnp.float32)
    n_proj = jnp.dot(n_ref[...], dk[64:80, :],
                     preferred_element_type=jnp.float32)
    l_proj = jnp.dot(l_ref[...], dk[80:112, :],
                     preferred_element_type=jnp.float32)
    num_proj = jnp.dot(num_ref[...], dk[112:144, :],
                       preferred_element_type=jnp.float32)
    p_proj = jnp.dot(p_ref[...], dk[144:176, :],
                     preferred_element_type=jnp.float32)
    w_proj = jnp.dot(w_ref[...], dk[176:208, :],
                     preferred_element_type=jnp.float32)

    idx = lax.broadcasted_iota(jnp.int32, (R2, 1), 0)
    nb = idx // 16
    lbit = (idx // 8) % 2
    nbit = (idx // 4) % 2
    pbit = (idx // 2) % 2
    wbit = idx % 2

    acc = b_ref[...]
    for k in range(5):
        acc = acc + jnp.where(nb == k, 1.0, 0.0) * n_proj[k:k + 1, :]
    acc = acc + jnp.where(lbit == 1, l_proj[1:2, :], l_proj[0:1, :])
    acc = acc + jnp.where(nbit == 1, num_proj[1:2, :], num_proj[0:1, :])
    acc = acc + jnp.where(pbit == 1, p_proj[1:2, :], p_proj[0:1, :])
    acc = acc + jnp.where(wbit == 1, w_proj[1:2, :], w_proj[0:1, :])
    t2_ref[...] = acc


def _prep_tables(c_table, n_table, l_table, num_table, p_table, w_table,
                 dense_kernel, dense_bias):
    return pl.pallas_call(
        _prep_body,
        out_shape=[
            jax.ShapeDtypeStruct((R1, D), jnp.float32),
            jax.ShapeDtypeStruct((R2, D), jnp.float32),
        ],
    )(c_table, n_table, l_table, num_table, p_table, w_table,
      dense_kernel, dense_bias.reshape(1, D))


def _sum_body(t1_ref, t2_ref, t12_ref):
    t12_ref[...] = t2_ref[...][:, None, :] + t1_ref[...][None, :, :]


def _sum_tables(t1, t2):
    return pl.pallas_call(
        _sum_body,
        grid=(R2 // 8,),
        in_specs=[
            pl.BlockSpec((R1, D), lambda j: (0, 0)),
            pl.BlockSpec((8, D), lambda j: (j, 0)),
        ],
        out_specs=pl.BlockSpec((8, R1, D), lambda j: (j, 0, 0)),
        out_shape=jax.ShapeDtypeStruct((R2, R1, D), jnp.float32),
    )(t1, t2)


def _sc_body(cc, nb, il, inum, ip, iw, t12, out,
             cc_v, nb_v, il_v, in_v, ip_v, iw_v, i_b,
             buf0, buf1, buf2, buf3, semIdx,
             g0, g1, g2, g3, o0, o1, o2, o3):
    wid = lax.axis_index("s") * NC + lax.axis_index("c")
    base0 = wid * NT

    def issue_gather(cl, buf, gsem):
        pltpu.async_copy(t12.at[i_b.at[pl.ds(cl * C, C)]], buf, gsem)

    def wait_gather(buf, gsem):
        pltpu.make_async_copy(t12.at[i_b.at[pl.ds(0, C)]], buf, gsem).wait()

    def writeout(cl, blk, buf, osem):
        base = base0 + blk * IB + cl * C
        pltpu.async_copy(buf, out.at[pl.ds(base, C)], osem)

    def wait_out(buf, osem):
        pltpu.make_async_copy(buf, out.at[pl.ds(0, C)], osem).wait()

    FULL = ((buf0, g0, o0), (buf1, g1, o1), (buf2, g2, o2), (buf3, g3, o3))
    SLOTS = tuple((b, o) for b, _, o in FULL)

    def block(blk, carry):
        bbase = base0 + blk * IB
        cps = [pltpu.async_copy(src.at[pl.ds(bbase, IB)], dst, semIdx)
               for src, dst in zip((cc, nb, il, inum, ip, iw),
                                   (cc_v, nb_v, il_v, in_v, ip_v, iw_v))]
        for cp in cps:
            cp.wait()

        def ixbody(j, carry2):
            sl = pl.ds(j * 16, 16)
            i_b[sl] = (nb_v[sl] * 16 + il_v[sl] * 8 + in_v[sl] * 4
                       + ip_v[sl] * 2 + iw_v[sl]) * R1 + lax.rem(cc_v[sl], R1)
            return carry2

        lax.fori_loop(0, IB // 16, ixbody, 0)

        @pl.when(blk > 0)
        def _():
            for buf, osem in SLOTS:
                wait_out(buf, osem)

        for m, (buf, gsem, osem) in enumerate(FULL):
            issue_gather(m, buf, gsem)

        def quad(q, carry2):
            c0 = 4 * q
            for m, (buf, gsem, osem) in enumerate(FULL):
                wait_gather(buf, gsem)
                writeout(c0 + m, blk, buf, osem)

            @pl.when(q < QB - 1)
            def _():
                for m, (buf, gsem, osem) in enumerate(FULL):
                    wait_out(buf, osem)
                    issue_gather(c0 + 4 + m, buf, gsem)

            return carry2

        lax.fori_loop(0, QB, quad, 0)
        return carry

    lax.fori_loop(0, NBLK, block, 0)
    for buf, osem in SLOTS:
        wait_out(buf, osem)


_sc_kernel = functools.partial(
    pl.kernel,
    mesh=plsc.VectorSubcoreMesh(core_axis_name="c", subcore_axis_name="s"),
    out_type=jax.ShapeDtypeStruct((N, D), jnp.float32),
    scratch_types=[
        pltpu.VMEM((IB,), jnp.int32),
        pltpu.VMEM((IB,), jnp.int32),
        pltpu.VMEM((IB,), jnp.int32),
        pltpu.VMEM((IB,), jnp.int32),
        pltpu.VMEM((IB,), jnp.int32),
        pltpu.VMEM((IB,), jnp.int32),
        pltpu.VMEM((IB,), jnp.int32),
        pltpu.VMEM((C, D), jnp.float32),
        pltpu.VMEM((C, D), jnp.float32),
        pltpu.VMEM((C, D), jnp.float32),
        pltpu.VMEM((C, D), jnp.float32),
        pltpu.SemaphoreType.DMA,
        pltpu.SemaphoreType.DMA,
        pltpu.SemaphoreType.DMA,
        pltpu.SemaphoreType.DMA,
        pltpu.SemaphoreType.DMA,
        pltpu.SemaphoreType.DMA,
        pltpu.SemaphoreType.DMA,
        pltpu.SemaphoreType.DMA,
        pltpu.SemaphoreType.DMA,
    ],
)(_sc_body)


def kernel(char_code, num_bytes, is_letter, is_number, is_punctuation,
           is_whitespace, c_table, n_table, l_table, num_table, p_table,
           w_table, dense_kernel, dense_bias):
    t1, t2 = _prep_tables(c_table, n_table, l_table, num_table, p_table,
                          w_table, dense_kernel, dense_bias)
    t12 = _sum_tables(t1, t2).reshape(R2 * R1, D)
    cc = char_code.reshape(N).astype(jnp.int32)
    nb = num_bytes.reshape(N).astype(jnp.int32)
    il = is_letter.reshape(N).astype(jnp.int32)
    inum = is_number.reshape(N).astype(jnp.int32)
    ip = is_punctuation.reshape(N).astype(jnp.int32)
    iw = is_whitespace.reshape(N).astype(jnp.int32)
    out = _sc_kernel(cc, nb, il, inum, ip, iw, t12)
    return out.reshape(B, P, H, W, D)
